# Initial kernel scaffold; baseline (speedup 1.0000x reference)
import jax
import jax.numpy as jnp
from jax.experimental import pallas as pl


def kernel(school_idx, goal_idx, method_idx, subject_pca, grade_pca, school_table, goal_table, method_table, subject_W, subject_b, grade_W, grade_b, W1, b1, W2, b2, W3, b3):
    raise NotImplementedError("write your pallas kernel here")


# speedup vs baseline: 1.1289x; 1.1289x over previous
import jax
import jax.numpy as jnp
from jax.experimental import pallas as pl
from jax.experimental.pallas import tpu as pltpu
from jax.experimental.pallas import tpu_sc as plsc

EMB = 64
NUM_WORKERS = 32
GATHER_CHUNK = 768
BATCH_BLOCK = 2048


def _sc_gather(table, idx):
    n = idx.shape[0]
    b_per_w = n // NUM_WORKERS
    n_chunks = b_per_w // GATHER_CHUNK
    mesh = plsc.VectorSubcoreMesh(core_axis_name="c", subcore_axis_name="s")

    @pl.kernel(
        out_type=jax.ShapeDtypeStruct((n, 128), table.dtype), mesh=mesh,
        scratch_types=[
            pltpu.VMEM((GATHER_CHUNK,), jnp.int32),
            pltpu.VMEM((GATHER_CHUNK, 128), jnp.float32),
            pltpu.SemaphoreType.DMA,
        ])
    def gather_kernel(table_hbm, idx_hbm, out_hbm, idx_v, rows_v, sem):
        wid = jax.lax.axis_index("s") * 2 + jax.lax.axis_index("c")
        base = wid * b_per_w
        for c in range(n_chunks):
            start = base + c * GATHER_CHUNK
            pltpu.sync_copy(idx_hbm.at[pl.ds(start, GATHER_CHUNK)], idx_v)
            pltpu.async_copy(table_hbm.at[idx_v], rows_v, sem).wait()
            pltpu.sync_copy(rows_v, out_hbm.at[pl.ds(start, GATHER_CHUNK)])

    return gather_kernel(table, idx)


def _tower_body(school_ref, goal_ref, method_ref, pca_ref,
                W1x_ref, Wp_ref, b1_ref, W2_ref, b2_ref, W3_ref, b3_ref,
                out_ref):
    f32 = jnp.float32
    x = jnp.concatenate(
        [school_ref[...], goal_ref[...], method_ref[...]], axis=-1)
    h = jnp.dot(x, W1x_ref[...], preferred_element_type=f32)
    h += jnp.dot(pca_ref[...], Wp_ref[...], preferred_element_type=f32)
    h = jnp.maximum(h + b1_ref[...], 0.0)
    h = jnp.dot(h, W2_ref[...], preferred_element_type=f32) + b2_ref[...]
    h = jnp.maximum(h, 0.0)
    out_ref[...] = jnp.dot(h, W3_ref[...],
                           preferred_element_type=f32) + b3_ref[...]


def kernel(school_idx, goal_idx, method_idx, subject_pca, grade_pca,
           school_table, goal_table, method_table,
           subject_W, subject_b, grade_W, grade_b,
           W1, b1, W2, b2, W3, b3):
    n = school_idx.shape[0]
    sv = school_table.shape[0]
    gv = goal_table.shape[0]

    pad = ((0, 0), (0, 128 - EMB))
    big_table = jnp.concatenate([
        jnp.pad(school_table, pad),
        jnp.pad(goal_table, pad),
        jnp.pad(method_table, pad),
    ], axis=0)
    big_idx = jnp.concatenate(
        [school_idx, goal_idx + sv, method_idx + sv + gv])
    gathered = _sc_gather(big_table, big_idx)

    z = jnp.zeros((128 - EMB, W1.shape[1]), W1.dtype)
    W1x = jnp.concatenate(
        [W1[0:64], z, W1[64:128], z, W1[128:192], z], axis=0)
    Wp = jnp.concatenate(
        [subject_W @ W1[192:224], grade_W @ W1[224:256]], axis=0)
    b1f = b1 + subject_b @ W1[192:224] + grade_b @ W1[224:256]
    pca = jnp.concatenate([subject_pca, grade_pca], axis=-1)

    bs = BATCH_BLOCK
    nblocks = n // bs

    def full_spec(shape):
        return pl.BlockSpec(shape, lambda i: (0,) * len(shape))

    out = pl.pallas_call(
        _tower_body,
        grid=(nblocks,),
        in_specs=[
            pl.BlockSpec((bs, 128), lambda i: (i, 0)),
            pl.BlockSpec((bs, 128), lambda i: (i + nblocks, 0)),
            pl.BlockSpec((bs, 128), lambda i: (i + 2 * nblocks, 0)),
            pl.BlockSpec((bs, pca.shape[1]), lambda i: (i, 0)),
            full_spec(W1x.shape), full_spec(Wp.shape),
            full_spec((1, b1f.shape[0])),
            full_spec(W2.shape), full_spec((1, b2.shape[0])),
            full_spec(W3.shape), full_spec((1, b3.shape[0])),
        ],
        out_specs=pl.BlockSpec((bs, W3.shape[1]), lambda i: (i, 0)),
        out_shape=jax.ShapeDtypeStruct((n, W3.shape[1]), jnp.float32),
    )(gathered, gathered, gathered, pca,
      W1x, Wp, b1f.reshape(1, -1),
      W2, b2.reshape(1, -1), W3, b3.reshape(1, -1))
    return out


# speedup vs baseline: 2.1644x; 1.9173x over previous
import jax
import jax.numpy as jnp
from jax.experimental import pallas as pl
from jax.experimental.pallas import tpu as pltpu
from jax.experimental.pallas import tpu_sc as plsc

EMB = 64
NUM_WORKERS = 32
BATCH_BLOCK = 2048


def _sc_gather3(school_table, goal_table, method_table,
                school_idx, goal_idx, method_idx):
    n = school_idx.shape[0]
    b_per_w = n // NUM_WORKERS
    sidx = school_idx.reshape(NUM_WORKERS, b_per_w)
    gidx = goal_idx.reshape(NUM_WORKERS, b_per_w)
    midx = method_idx.reshape(NUM_WORKERS, b_per_w)
    out_t = jax.ShapeDtypeStruct((n, 128), jnp.float32)
    mesh = plsc.VectorSubcoreMesh(core_axis_name="c", subcore_axis_name="s")

    @pl.kernel(
        out_type=[out_t, out_t, out_t], mesh=mesh,
        scratch_types=[
            pltpu.VMEM((b_per_w,), jnp.int32),
            pltpu.VMEM((b_per_w, 128), jnp.float32),
            pltpu.SemaphoreType.DMA,
        ])
    def gather_kernel(school_hbm, goal_hbm, method_hbm,
                      si_hbm, gi_hbm, mi_hbm,
                      so_hbm, go_hbm, mo_hbm,
                      idx_v, rows_v, sem):
        wid = jax.lax.axis_index("s") * 2 + jax.lax.axis_index("c")
        base = wid * b_per_w
        for table_hbm, i_hbm, o_hbm in ((school_hbm, si_hbm, so_hbm),
                                        (goal_hbm, gi_hbm, go_hbm),
                                        (method_hbm, mi_hbm, mo_hbm)):
            pltpu.sync_copy(i_hbm.at[wid], idx_v)
            pltpu.async_copy(table_hbm.at[idx_v], rows_v, sem).wait()
            pltpu.sync_copy(rows_v, o_hbm.at[pl.ds(base, b_per_w)])

    return gather_kernel(school_table, goal_table, method_table,
                         sidx, gidx, midx)


def _tower_body(school_ref, goal_ref, method_ref, pca_ref,
                W1x_ref, Wp_ref, b1_ref, W2_ref, b2_ref, W3_ref, b3_ref,
                out_ref):
    f32 = jnp.float32
    bf16 = jnp.bfloat16
    x = jnp.concatenate(
        [school_ref[...], goal_ref[...], method_ref[...]],
        axis=-1).astype(bf16)
    h = jnp.dot(x, W1x_ref[...].astype(bf16), preferred_element_type=f32)
    h += jnp.dot(pca_ref[...].astype(bf16), Wp_ref[...].astype(bf16),
                 preferred_element_type=f32)
    h = jnp.maximum(h + b1_ref[...], 0.0).astype(bf16)
    h = jnp.dot(h, W2_ref[...].astype(bf16), preferred_element_type=f32)
    h = jnp.maximum(h + b2_ref[...], 0.0).astype(bf16)
    out_ref[...] = jnp.dot(h, W3_ref[...].astype(bf16),
                           preferred_element_type=f32) + b3_ref[...]


def kernel(school_idx, goal_idx, method_idx, subject_pca, grade_pca,
           school_table, goal_table, method_table,
           subject_W, subject_b, grade_W, grade_b,
           W1, b1, W2, b2, W3, b3):
    n = school_idx.shape[0]
    pad = ((0, 0), (0, 128 - EMB))
    school_emb, goal_emb, method_emb = _sc_gather3(
        jnp.pad(school_table, pad), jnp.pad(goal_table, pad),
        jnp.pad(method_table, pad),
        school_idx, goal_idx, method_idx)

    z = jnp.zeros((128 - EMB, W1.shape[1]), W1.dtype)
    W1x = jnp.concatenate(
        [W1[0:64], z, W1[64:128], z, W1[128:192], z], axis=0)
    Wp = jnp.concatenate(
        [subject_W @ W1[192:224], grade_W @ W1[224:256]], axis=0)
    b1f = b1 + subject_b @ W1[192:224] + grade_b @ W1[224:256]
    pca = jnp.concatenate([subject_pca, grade_pca], axis=-1)

    bs = BATCH_BLOCK

    def full_spec(shape):
        return pl.BlockSpec(shape, lambda i: (0,) * len(shape))

    def batch_spec(cols):
        return pl.BlockSpec((bs, cols), lambda i: (i, 0))

    out = pl.pallas_call(
        _tower_body,
        grid=(n // bs,),
        in_specs=[
            batch_spec(128), batch_spec(128), batch_spec(128),
            batch_spec(pca.shape[1]),
            full_spec(W1x.shape), full_spec(Wp.shape),
            full_spec((1, b1f.shape[0])),
            full_spec(W2.shape), full_spec((1, b2.shape[0])),
            full_spec(W3.shape), full_spec((1, b3.shape[0])),
        ],
        out_specs=batch_spec(W3.shape[1]),
        out_shape=jax.ShapeDtypeStruct((n, W3.shape[1]), jnp.float32),
    )(school_emb, goal_emb, method_emb, pca,
      W1x, Wp, b1f.reshape(1, -1),
      W2, b2.reshape(1, -1), W3, b3.reshape(1, -1))
    return out


# speedup vs baseline: 2.2318x; 1.0311x over previous
import jax
import jax.numpy as jnp
from jax.experimental import pallas as pl
from jax.experimental.pallas import tpu as pltpu
from jax.experimental.pallas import tpu_sc as plsc

EMB = 64
NUM_WORKERS = 32
BATCH_BLOCK = 4096


def _sc_gather3(school_table, goal_table, method_table,
                school_idx, goal_idx, method_idx):
    n = school_idx.shape[0]
    b_per_w = n // NUM_WORKERS
    ch = b_per_w // 2
    sidx = school_idx.reshape(NUM_WORKERS, b_per_w)
    gidx = goal_idx.reshape(NUM_WORKERS, b_per_w)
    midx = method_idx.reshape(NUM_WORKERS, b_per_w)
    out_t = jax.ShapeDtypeStruct((n, 128), jnp.float32)
    row_buf = pltpu.VMEM((ch, 128), jnp.float32)
    idx_buf = pltpu.VMEM((ch,), jnp.int32)
    mesh = plsc.VectorSubcoreMesh(core_axis_name="c", subcore_axis_name="s")

    @pl.kernel(
        out_type=[out_t, out_t, out_t], mesh=mesh,
        scratch_types=[
            idx_buf, idx_buf, row_buf, row_buf,
            pltpu.SemaphoreType.DMA, pltpu.SemaphoreType.DMA,
            pltpu.SemaphoreType.DMA, pltpu.SemaphoreType.DMA,
        ])
    def gather_kernel(school_hbm, goal_hbm, method_hbm,
                      si_hbm, gi_hbm, mi_hbm,
                      so_hbm, go_hbm, mo_hbm,
                      i0, i1, r0, r1, g0, g1, w0, w1):
        wid = jax.lax.axis_index("s") * 2 + jax.lax.axis_index("c")
        base = wid * b_per_w
        tables = ((school_hbm, si_hbm, so_hbm),
                  (goal_hbm, gi_hbm, go_hbm),
                  (method_hbm, mi_hbm, mo_hbm))
        items = [(t, c) for t in range(3) for c in range(2)]
        ibufs, rbufs, gsems, wsems = (i0, i1), (r0, r1), (g0, g1), (w0, w1)
        gathers = [None, None]
        writes = [None, None]
        for k, (t, c) in enumerate(items):
            b = k % 2
            table_hbm, i_hbm, o_hbm = tables[t]
            if writes[b] is not None:
                writes[b].wait()
            pltpu.sync_copy(i_hbm.at[wid, pl.ds(c * ch, ch)], ibufs[b])
            gathers[b] = pltpu.async_copy(
                table_hbm.at[ibufs[b]], rbufs[b], gsems[b])
            pb = 1 - b
            if gathers[pb] is not None:
                gathers[pb].wait()
                pt, pc = items[k - 1]
                writes[pb] = pltpu.async_copy(
                    rbufs[pb],
                    tables[pt][2].at[pl.ds(base + pc * ch, ch)], wsems[pb])
                gathers[pb] = None
        gathers[1].wait()
        writes[1] = pltpu.async_copy(
            rbufs[1], tables[2][2].at[pl.ds(base + ch, ch)], wsems[1])
        writes[0].wait()
        writes[1].wait()

    return gather_kernel(school_table, goal_table, method_table,
                         sidx, gidx, midx)


def _tower_body(school_ref, goal_ref, method_ref, pca_ref,
                W1x_ref, Wp_ref, b1_ref, W2_ref, b2_ref, W3_ref, b3_ref,
                out_ref):
    f32 = jnp.float32
    bf16 = jnp.bfloat16
    x = jnp.concatenate(
        [school_ref[...], goal_ref[...], method_ref[...]],
        axis=-1).astype(bf16)
    h = jnp.dot(x, W1x_ref[...], preferred_element_type=f32)
    h += jnp.dot(pca_ref[...].astype(bf16), Wp_ref[...],
                 preferred_element_type=f32)
    h = jnp.maximum(h + b1_ref[...], 0.0).astype(bf16)
    h = jnp.dot(h, W2_ref[...], preferred_element_type=f32)
    h = jnp.maximum(h + b2_ref[...], 0.0).astype(bf16)
    out_ref[...] = jnp.dot(h, W3_ref[...],
                           preferred_element_type=f32) + b3_ref[...]


def kernel(school_idx, goal_idx, method_idx, subject_pca, grade_pca,
           school_table, goal_table, method_table,
           subject_W, subject_b, grade_W, grade_b,
           W1, b1, W2, b2, W3, b3):
    n = school_idx.shape[0]
    bf16 = jnp.bfloat16
    pad = ((0, 0), (0, 128 - EMB))
    school_emb, goal_emb, method_emb = _sc_gather3(
        jnp.pad(school_table, pad),
        jnp.pad(goal_table, pad),
        jnp.pad(method_table, pad),
        school_idx, goal_idx, method_idx)

    z = jnp.zeros((128 - EMB, W1.shape[1]), W1.dtype)
    W1x = jnp.concatenate(
        [W1[0:64], z, W1[64:128], z, W1[128:192], z], axis=0).astype(bf16)
    Wp = jnp.concatenate(
        [subject_W @ W1[192:224], grade_W @ W1[224:256]],
        axis=0).astype(bf16)
    b1f = b1 + subject_b @ W1[192:224] + grade_b @ W1[224:256]
    pca = jnp.concatenate([subject_pca, grade_pca], axis=-1)

    bs = BATCH_BLOCK

    def full_spec(shape):
        return pl.BlockSpec(shape, lambda i: (0,) * len(shape))

    def batch_spec(cols):
        return pl.BlockSpec((bs, cols), lambda i: (i, 0))

    out = pl.pallas_call(
        _tower_body,
        grid=(n // bs,),
        in_specs=[
            batch_spec(128), batch_spec(128), batch_spec(128),
            batch_spec(pca.shape[1]),
            full_spec(W1x.shape), full_spec(Wp.shape),
            full_spec((1, b1f.shape[0])),
            full_spec(W2.shape), full_spec((1, b2.shape[0])),
            full_spec(W3.shape), full_spec((1, b3.shape[0])),
        ],
        out_specs=batch_spec(W3.shape[1]),
        out_shape=jax.ShapeDtypeStruct((n, W3.shape[1]), jnp.float32),
    )(school_emb, goal_emb, method_emb, pca,
      W1x, Wp, b1f.reshape(1, -1),
      W2.astype(bf16), b2.reshape(1, -1),
      W3.astype(bf16), b3.reshape(1, -1))
    return out
